# baseline (speedup 1.0000x reference)
import functools

import jax
import jax.numpy as jnp
from jax import lax
from jax.experimental import pallas as pl
from jax.experimental.pallas import tpu as pltpu
from jax.experimental.pallas import tpu_sc as plsc

N, C, P, M = 32, 128, 16384, 4096
L = 16
NC, NS = 2, 16
R = 1
NBLK = C // R


def _gather_body(points_hbm, idx_hbm, out_hbm, idx_v, row_v0, row_v1,
                 row_v2, row_v3, out_v0, out_v1, out_v2, out_v3,
                 in_sem0, in_sem1, in_sem2, in_sem3,
                 out_sem0, out_sem1, out_sem2, out_sem3):
    n = lax.axis_index("s") * NC + lax.axis_index("c")
    rows = (row_v0, row_v1, row_v2, row_v3)
    outs = (out_v0, out_v1, out_v2, out_v3)
    in_sems = (in_sem0, in_sem1, in_sem2, in_sem3)
    out_sems = (out_sem0, out_sem1, out_sem2, out_sem3)

    pltpu.sync_copy(idx_hbm.at[n], idx_v)

    for b in range(4):
        for r in range(R):
            pltpu.async_copy(
                points_hbm.at[n, b * R + r], rows[b].at[pl.ds(r * P, P)],
                in_sems[b])

    def outer(j0, carry):
        for b in range(4):
            j = j0 * 4 + b
            for r in range(R):
                pltpu.make_async_copy(
                    points_hbm.at[n, j * R + r],
                    rows[b].at[pl.ds(r * P, P)], in_sems[b]).wait()

            @pl.when(j0 > 0)
            def _wait_out():
                for r in range(R):
                    pltpu.make_async_copy(
                        outs[b].at[pl.ds(r * M, M)],
                        out_hbm.at[n, (j - 4) * R + r], out_sems[b]).wait()

            @plsc.parallel_loop(0, M, L, unroll=16)
            def g_loop(i):
                base = pl.multiple_of(i, L)
                idx = idx_v[pl.ds(base, L)]
                for r in range(R):
                    ridx = idx + (r * P) if r else idx
                    outs[b][pl.ds(r * M + base, L)] = plsc.load_gather(
                        rows[b], [ridx])

            for r in range(R):
                pltpu.async_copy(
                    outs[b].at[pl.ds(r * M, M)], out_hbm.at[n, j * R + r],
                    out_sems[b])

            @pl.when(j + 4 < NBLK)
            def _next_in():
                for r in range(R):
                    pltpu.async_copy(
                        points_hbm.at[n, (j + 4) * R + r],
                        rows[b].at[pl.ds(r * P, P)], in_sems[b])
        return carry

    lax.fori_loop(0, NBLK // 4, outer, 0)

    for b in range(4):
        for r in range(R):
            pltpu.make_async_copy(
                outs[b].at[pl.ds(r * M, M)],
                out_hbm.at[n, (NBLK - 4 + b) * R + r], out_sems[b]).wait()


@jax.jit
def kernel(points, indices):
    idx32 = indices.astype(jnp.int32)
    mesh = plsc.VectorSubcoreMesh(core_axis_name="c", subcore_axis_name="s")
    run = functools.partial(
        pl.kernel,
        mesh=mesh,
        out_type=jax.ShapeDtypeStruct((N, C, M), jnp.float32),
        scratch_types=[
            pltpu.VMEM((M,), jnp.int32),
            pltpu.VMEM((R * P,), jnp.float32),
            pltpu.VMEM((R * P,), jnp.float32),
            pltpu.VMEM((R * P,), jnp.float32),
            pltpu.VMEM((R * P,), jnp.float32),
            pltpu.VMEM((R * M,), jnp.float32),
            pltpu.VMEM((R * M,), jnp.float32),
            pltpu.VMEM((R * M,), jnp.float32),
            pltpu.VMEM((R * M,), jnp.float32),
            pltpu.SemaphoreType.DMA,
            pltpu.SemaphoreType.DMA,
            pltpu.SemaphoreType.DMA,
            pltpu.SemaphoreType.DMA,
            pltpu.SemaphoreType.DMA,
            pltpu.SemaphoreType.DMA,
            pltpu.SemaphoreType.DMA,
            pltpu.SemaphoreType.DMA,
        ],
        compiler_params=pltpu.CompilerParams(needs_layout_passes=False),
    )(_gather_body)
    return run(points, idx32)


# speedup vs baseline: 1.0049x; 1.0049x over previous
import functools

import jax
import jax.numpy as jnp
from jax import lax
from jax.experimental import pallas as pl
from jax.experimental.pallas import tpu as pltpu
from jax.experimental.pallas import tpu_sc as plsc

N, C, P, M = 32, 128, 16384, 4096
L = 16
NC, NS = 2, 16
R = 1
NBLK = C // R


def _gather_body(points_hbm, idx_hbm, out_hbm, idx_v, row_v0, row_v1,
                 row_v2, row_v3, out_v0, out_v1, out_v2, out_v3,
                 in_sem0, in_sem1, in_sem2, in_sem3,
                 out_sem0, out_sem1, out_sem2, out_sem3):
    n = lax.axis_index("s") * NC + lax.axis_index("c")
    rows = (row_v0, row_v1, row_v2, row_v3)
    outs = (out_v0, out_v1, out_v2, out_v3)
    in_sems = (in_sem0, in_sem1, in_sem2, in_sem3)
    out_sems = (out_sem0, out_sem1, out_sem2, out_sem3)

    pltpu.sync_copy(idx_hbm.at[n], idx_v)

    for b in range(4):
        for r in range(R):
            pltpu.async_copy(
                points_hbm.at[n, b * R + r], rows[b].at[pl.ds(r * P, P)],
                in_sems[b])

    def outer(j0, carry):
        for b in range(4):
            j = j0 * 4 + b
            for r in range(R):
                pltpu.make_async_copy(
                    points_hbm.at[n, j * R + r],
                    rows[b].at[pl.ds(r * P, P)], in_sems[b]).wait()

            @pl.when(j0 > 0)
            def _wait_out():
                for r in range(R):
                    pltpu.make_async_copy(
                        outs[b].at[pl.ds(r * M, M)],
                        out_hbm.at[n, (j - 4) * R + r], out_sems[b]).wait()

            @plsc.parallel_loop(0, M, L, unroll=8)
            def g_loop(i):
                base = pl.multiple_of(i, L)
                idx = idx_v[pl.ds(base, L)]
                for r in range(R):
                    ridx = idx + (r * P) if r else idx
                    outs[b][pl.ds(r * M + base, L)] = plsc.load_gather(
                        rows[b], [ridx])

            for r in range(R):
                pltpu.async_copy(
                    outs[b].at[pl.ds(r * M, M)], out_hbm.at[n, j * R + r],
                    out_sems[b])

            @pl.when(j + 4 < NBLK)
            def _next_in():
                for r in range(R):
                    pltpu.async_copy(
                        points_hbm.at[n, (j + 4) * R + r],
                        rows[b].at[pl.ds(r * P, P)], in_sems[b])
        return carry

    lax.fori_loop(0, NBLK // 4, outer, 0)

    for b in range(4):
        for r in range(R):
            pltpu.make_async_copy(
                outs[b].at[pl.ds(r * M, M)],
                out_hbm.at[n, (NBLK - 4 + b) * R + r], out_sems[b]).wait()


@jax.jit
def kernel(points, indices):
    idx32 = indices.astype(jnp.int32)
    mesh = plsc.VectorSubcoreMesh(core_axis_name="c", subcore_axis_name="s")
    run = functools.partial(
        pl.kernel,
        mesh=mesh,
        out_type=jax.ShapeDtypeStruct((N, C, M), jnp.float32),
        scratch_types=[
            pltpu.VMEM((M,), jnp.int32),
            pltpu.VMEM((R * P,), jnp.float32),
            pltpu.VMEM((R * P,), jnp.float32),
            pltpu.VMEM((R * P,), jnp.float32),
            pltpu.VMEM((R * P,), jnp.float32),
            pltpu.VMEM((R * M,), jnp.float32),
            pltpu.VMEM((R * M,), jnp.float32),
            pltpu.VMEM((R * M,), jnp.float32),
            pltpu.VMEM((R * M,), jnp.float32),
            pltpu.SemaphoreType.DMA,
            pltpu.SemaphoreType.DMA,
            pltpu.SemaphoreType.DMA,
            pltpu.SemaphoreType.DMA,
            pltpu.SemaphoreType.DMA,
            pltpu.SemaphoreType.DMA,
            pltpu.SemaphoreType.DMA,
            pltpu.SemaphoreType.DMA,
        ],
        compiler_params=pltpu.CompilerParams(needs_layout_passes=False),
    )(_gather_body)
    return run(points, idx32)
